# Initial kernel scaffold; baseline (speedup 1.0000x reference)
import jax
import jax.numpy as jnp
from jax.experimental import pallas as pl


def kernel(feat, coordinate, edge_index, params):
    raise NotImplementedError("write your pallas kernel here")


# speedup vs baseline: 2.7273x; 2.7273x over previous
import functools

import jax
import jax.numpy as jnp
from jax import lax
from jax.experimental import pallas as pl
from jax.experimental.pallas import tpu as pltpu
from jax.experimental.pallas import tpu_sc as plsc

N_NODES = 10000
N_EDGES = 320000
D = 128
XPAD = 16
INV_DEG = 0.1

BLK_E = 2000
BLK_N = 2000


def _silu(v):
    return v * jax.nn.sigmoid(v)


def _dot(a, b):
    return jnp.dot(a, b, preferred_element_type=jnp.float32)


def _bf16_round(v):
    return v.astype(jnp.bfloat16).astype(jnp.float32)


def _entry_body(feat, w_emb, b_emb, w_d, b1, w_s, h0, pre_d, pre_s):
    h = _dot(feat[...], w_emb[...]) + b_emb[...]
    h0[...] = h
    pre_d[...] = _dot(h, w_d[...]) + b1[...]
    pre_s[...] = _dot(h, w_s[...])


def _entry_call(feat, w_emb, b_emb, w_d, b1, w_s):
    nb = N_NODES // BLK_N
    row_spec = pl.BlockSpec((BLK_N, D), lambda i: (i, 0))
    w_spec = pl.BlockSpec((D, D), lambda i: (0, 0))
    b_spec = pl.BlockSpec((1, D), lambda i: (0, 0))
    out = jax.ShapeDtypeStruct((N_NODES, D), jnp.float32)
    return pl.pallas_call(
        _entry_body,
        grid=(nb,),
        in_specs=[row_spec, w_spec, b_spec, w_spec, b_spec, w_spec],
        out_specs=[row_spec, row_spec, row_spec],
        out_shape=[out, out, out],
    )(feat, w_emb, b_emb, w_d, b1, w_s)


def _edge_body(spre, diff, wdist, ew2, eb2, xw1, xb1, xw2r, xb2, m2s, dw):
    d = diff[...]
    dist2 = jnp.sum(d * d, axis=1, keepdims=True)
    m = _silu(spre[...] + dist2 * wdist[...])
    m2 = _silu(_dot(m, ew2[...]) + eb2[...])
    t = _silu(_dot(m2, xw1[...]) + xb1[...])
    w = jnp.sum(t * xw2r[...], axis=1, keepdims=True) + xb2[...]
    m2s[...] = m2
    dw[...] = d * w


def _edge_call(spre, diff, wdist, ew2, eb2, xw1, xb1, xw2r, xb2):
    nb = N_EDGES // BLK_E
    row_spec = pl.BlockSpec((BLK_E, D), lambda i: (i, 0))
    x_spec = pl.BlockSpec((BLK_E, XPAD), lambda i: (i, 0))
    w_spec = pl.BlockSpec((D, D), lambda i: (0, 0))
    b_spec = pl.BlockSpec((1, D), lambda i: (0, 0))
    s_spec = pl.BlockSpec((1, 1), lambda i: (0, 0))
    return pl.pallas_call(
        _edge_body,
        grid=(nb,),
        in_specs=[row_spec, x_spec, b_spec, w_spec, b_spec, w_spec, b_spec,
                  b_spec, s_spec],
        out_specs=[row_spec, x_spec],
        out_shape=[jax.ShapeDtypeStruct((N_EDGES, D), jnp.float32),
                   jax.ShapeDtypeStruct((N_EDGES, XPAD), jnp.float32)],
    )(spre, diff, wdist, ew2, eb2, xw1, xb1, xw2r, xb2)


def _node_mid_body(h, xp, agg0, agg1, dx0, dx1, hw1a, hw1b, hb1, hw2, hb2,
                   w_d, b1n, w_s, h2o, xp2o, pre_d, pre_s):
    agg = (agg0[...] + agg1[...]) / 10.0
    u = _silu(_dot(h[...], hw1a[...]) + _dot(agg, hw1b[...]) + hb1[...])
    h2 = h[...] + _dot(u, hw2[...]) + hb2[...]
    h2o[...] = h2
    xp2o[...] = xp[...] + (dx0[...] + dx1[...]) / 10.0
    pre_d[...] = _dot(h2, w_d[...]) + b1n[...]
    pre_s[...] = _dot(h2, w_s[...])


def _node_mid_call(h, xp, agg0, agg1, dx0, dx1, hw1a, hw1b, hb1, hw2, hb2,
                   w_d, b1n, w_s):
    nb = N_NODES // BLK_N
    row_spec = pl.BlockSpec((BLK_N, D), lambda i: (i, 0))
    x_spec = pl.BlockSpec((BLK_N, XPAD), lambda i: (i, 0))
    w_spec = pl.BlockSpec((D, D), lambda i: (0, 0))
    b_spec = pl.BlockSpec((1, D), lambda i: (0, 0))
    outd = jax.ShapeDtypeStruct((N_NODES, D), jnp.float32)
    outx = jax.ShapeDtypeStruct((N_NODES, XPAD), jnp.float32)
    return pl.pallas_call(
        _node_mid_body,
        grid=(nb,),
        in_specs=[row_spec, x_spec, row_spec, row_spec, x_spec, x_spec,
                  w_spec, w_spec, b_spec, w_spec, b_spec,
                  w_spec, b_spec, w_spec],
        out_specs=[row_spec, x_spec, row_spec, row_spec],
        out_shape=[outd, outx, outd, outd],
    )(h, xp, agg0, agg1, dx0, dx1, hw1a, hw1b, hb1, hw2, hb2, w_d, b1n, w_s)


def _node_fin_body(h, xp, agg0, agg1, dx0, dx1, hw1a, hw1b, hb1, hw2, hb2,
                   w_out, b_out, outo, xp2o):
    agg = (agg0[...] + agg1[...]) / 10.0
    u = _silu(_dot(h[...], hw1a[...]) + _dot(agg, hw1b[...]) + hb1[...])
    h2 = h[...] + _dot(u, hw2[...]) + hb2[...]
    outo[...] = _dot(h2, w_out[...]) + b_out[...]
    xp2o[...] = xp[...] + (dx0[...] + dx1[...]) / 10.0


def _node_fin_call(h, xp, agg0, agg1, dx0, dx1, hw1a, hw1b, hb1, hw2, hb2,
                   w_out, b_out):
    nb = N_NODES // BLK_N
    row_spec = pl.BlockSpec((BLK_N, D), lambda i: (i, 0))
    x_spec = pl.BlockSpec((BLK_N, XPAD), lambda i: (i, 0))
    w_spec = pl.BlockSpec((D, D), lambda i: (0, 0))
    b_spec = pl.BlockSpec((1, D), lambda i: (0, 0))
    return pl.pallas_call(
        _node_fin_body,
        grid=(nb,),
        in_specs=[row_spec, x_spec, row_spec, row_spec, x_spec, x_spec,
                  w_spec, w_spec, b_spec, w_spec, b_spec, w_spec, b_spec],
        out_specs=[row_spec, x_spec],
        out_shape=[jax.ShapeDtypeStruct((N_NODES, D), jnp.float32),
                   jax.ShapeDtypeStruct((N_NODES, XPAD), jnp.float32)],
    )(h, xp, agg0, agg1, dx0, dx1, hw1a, hw1b, hb1, hw2, hb2, w_out, b_out)


_SC_MESH = plsc.VectorSubcoreMesh(core_axis_name="c", subcore_axis_name="s")
NW = 32
EPW = N_EDGES // NW
CH = 80
NCH = EPW // CH


def _wid():
    return lax.axis_index("s") * 2 + lax.axis_index("c")


_SC_PARAMS = pltpu.CompilerParams(use_tc_tiling_on_sc=False)


def _sc_gather_body(pre_d, pre_s, xp, src, dst, s_pre_o, diff_o,
                    ids, idd, pd_b, ps_b, xd_b, xs_b, sem):
    w = _wid()

    def chunk(t, _):
        base = w * EPW + t * CH
        pltpu.sync_copy(src.at[pl.ds(base, CH)], ids)
        pltpu.sync_copy(dst.at[pl.ds(base, CH)], idd)
        cp1 = pltpu.async_copy(pre_d.at[idd], pd_b, sem)
        cp2 = pltpu.async_copy(pre_s.at[ids], ps_b, sem)
        cp3 = pltpu.async_copy(xp.at[idd], xd_b, sem)
        cp4 = pltpu.async_copy(xp.at[ids], xs_b, sem)
        cp1.wait()
        cp2.wait()
        cp3.wait()
        cp4.wait()

        def row(i, _):
            for j in range(D // 16):
                sl = pl.ds(j * 16, 16)
                pd_b[i, sl] = pd_b[i, sl] + ps_b[i, sl]
            xd_b[i, :] = xd_b[i, :] - xs_b[i, :]
            return 0

        lax.fori_loop(0, CH, row, 0)
        pltpu.sync_copy(pd_b, s_pre_o.at[pl.ds(base, CH)])
        pltpu.sync_copy(xd_b, diff_o.at[pl.ds(base, CH)])
        return 0

    lax.fori_loop(0, NCH, chunk, 0)


@functools.partial(
    pl.kernel,
    out_type=[jax.ShapeDtypeStruct((N_EDGES, D), jnp.float32),
              jax.ShapeDtypeStruct((N_EDGES, XPAD), jnp.float32)],
    mesh=_SC_MESH,
    compiler_params=_SC_PARAMS,
    scratch_types=[
        pltpu.VMEM((CH,), jnp.int32),
        pltpu.VMEM((CH,), jnp.int32),
        pltpu.VMEM((CH, D), jnp.float32),
        pltpu.VMEM((CH, D), jnp.float32),
        pltpu.VMEM((CH, XPAD), jnp.float32),
        pltpu.VMEM((CH, XPAD), jnp.float32),
        pltpu.SemaphoreType.DMA,
    ],
)
def _sc_gather_kernel(*args):
    _sc_gather_body(*args)


def _sc_gather(pre_d, pre_s, xp, src, dst):
    return _sc_gather_kernel(pre_d, pre_s, xp, src, dst)


NTILE = 16
EPC = N_EDGES // 2
EPT = EPC // NTILE
NRT = N_NODES // NTILE
ZR = 125


def _sc_scatter_body(m2, dw, dst, agg_o, dx_o,
                     idd, m2_b, dw_b, z128, z16, agg_sh, dx_sh, sem):
    cid = lax.axis_index("c")
    sid = lax.axis_index("s")

    def zrow(i, _):
        for j in range(D // 16):
            z128[i, pl.ds(j * 16, 16)] = jnp.zeros((16,), jnp.float32)
        z16[i, :] = jnp.zeros((16,), jnp.float32)
        return 0

    lax.fori_loop(0, ZR, zrow, 0)
    for k in range(NRT // ZR):
        off = sid * NRT + k * ZR
        pltpu.sync_copy(z128, agg_sh.at[pl.ds(off, ZR)])
        pltpu.sync_copy(z16, dx_sh.at[pl.ds(off, ZR)])
    plsc.subcore_barrier()

    def chunk(t, _):
        base = cid * EPC + sid * EPT + t * CH
        pltpu.sync_copy(dst.at[pl.ds(base, CH)], idd)
        cp1 = pltpu.async_copy(m2.at[pl.ds(base, CH)], m2_b, sem)
        cp2 = pltpu.async_copy(dw.at[pl.ds(base, CH)], dw_b, sem)
        cp1.wait()
        cp2.wait()
        pltpu.sync_copy(m2_b, agg_sh.at[idd], add=True)
        pltpu.sync_copy(dw_b, dx_sh.at[idd], add=True)
        return 0

    lax.fori_loop(0, EPT // CH, chunk, 0)
    plsc.subcore_barrier()

    for k in range(NRT // ZR):
        off = sid * NRT + k * ZR
        pltpu.sync_copy(agg_sh.at[pl.ds(off, ZR)], z128)
        pltpu.sync_copy(dx_sh.at[pl.ds(off, ZR)], z16)
        pltpu.sync_copy(z128, agg_o.at[cid, pl.ds(off, ZR)])
        pltpu.sync_copy(z16, dx_o.at[cid, pl.ds(off, ZR)])


@functools.partial(
    pl.kernel,
    out_type=[jax.ShapeDtypeStruct((2, N_NODES, D), jnp.float32),
              jax.ShapeDtypeStruct((2, N_NODES, XPAD), jnp.float32)],
    mesh=_SC_MESH,
    compiler_params=_SC_PARAMS,
    scratch_types=[
        pltpu.VMEM((CH,), jnp.int32),
        pltpu.VMEM((CH, D), jnp.float32),
        pltpu.VMEM((CH, XPAD), jnp.float32),
        pltpu.VMEM((ZR, D), jnp.float32),
        pltpu.VMEM((ZR, XPAD), jnp.float32),
        pltpu.VMEM_SHARED((N_NODES, D), jnp.float32),
        pltpu.VMEM_SHARED((N_NODES, XPAD), jnp.float32),
        pltpu.SemaphoreType.DMA,
    ],
)
def _sc_scatter_kernel(*args):
    _sc_scatter_body(*args)


def _sc_scatter(m2s, dw, dst):
    agg, dx = _sc_scatter_kernel(m2s, dw, dst)
    return agg[0], agg[1], dx[0], dx[1]


def kernel(feat, coordinate, edge_index, params):
    src = edge_index[0].astype(jnp.int32)
    dst = edge_index[1].astype(jnp.int32)
    xp = jnp.pad(coordinate, ((0, 0), (0, XPAD - 3)))

    lp = params['layers']

    def wparts(p):
        w1 = p['e_w1']
        return (w1[:D], w1[D:2 * D], w1[2 * D:2 * D + 1],
                p['e_b1'][None, :])

    w_d0, w_s0, _, b10 = wparts(lp[0])
    h, pre_d, pre_s = _entry_call(
        feat, params['emb_in_w'], params['emb_in_b'][None, :],
        w_d0, b10, w_s0)

    for li, p in enumerate(lp):
        _, _, wdist, _ = wparts(p)
        s_pre, diff = _sc_gather(pre_d, pre_s, xp, src, dst)
        m2s, dw = _edge_call(
            s_pre, diff, wdist, p['e_w2'], p['e_b2'][None, :],
            p['x_w1'], p['x_b1'][None, :], p['x_w2'].T, p['x_b2'][None, :])
        agg0, agg1, dx0, dx1 = _sc_scatter(m2s, dw, dst)
        hw1a, hw1b = p['h_w1'][:D], p['h_w1'][D:]
        if li + 1 < len(lp):
            w_dn, w_sn, _, b1n = wparts(lp[li + 1])
            h, xp, pre_d, pre_s = _node_mid_call(
                h, xp, agg0, agg1, dx0, dx1,
                hw1a, hw1b, p['h_b1'][None, :], p['h_w2'], p['h_b2'][None, :],
                w_dn, b1n, w_sn)
        else:
            out, xp = _node_fin_call(
                h, xp, agg0, agg1, dx0, dx1,
                hw1a, hw1b, p['h_b1'][None, :], p['h_w2'], p['h_b2'][None, :],
                params['emb_out_w'], params['emb_out_b'][None, :])

    return out, xp[:, :3]


# speedup vs baseline: 3.1109x; 1.1406x over previous
import functools

import jax
import jax.numpy as jnp
from jax import lax
from jax.experimental import pallas as pl
from jax.experimental.pallas import tpu as pltpu
from jax.experimental.pallas import tpu_sc as plsc

N_NODES = 10000
N_EDGES = 320000
D = 128
XPAD = 16
INV_DEG = 0.1

BLK_E = 2000
BLK_N = 2000


def _silu(v):
    return v * jax.nn.sigmoid(v)


def _dot(a, b):
    return jnp.dot(a, b, preferred_element_type=jnp.float32)


def _bf16_round(v):
    return v.astype(jnp.bfloat16).astype(jnp.float32)


def _entry_body(feat, w_emb, b_emb, w_d, b1, w_s, h0, pre_d, pre_s):
    h = _dot(feat[...], w_emb[...]) + b_emb[...]
    h0[...] = h
    pre_d[...] = _dot(h, w_d[...]) + b1[...]
    pre_s[...] = _dot(h, w_s[...])


def _entry_call(feat, w_emb, b_emb, w_d, b1, w_s):
    nb = N_NODES // BLK_N
    row_spec = pl.BlockSpec((BLK_N, D), lambda i: (i, 0))
    w_spec = pl.BlockSpec((D, D), lambda i: (0, 0))
    b_spec = pl.BlockSpec((1, D), lambda i: (0, 0))
    out = jax.ShapeDtypeStruct((N_NODES, D), jnp.float32)
    return pl.pallas_call(
        _entry_body,
        grid=(nb,),
        in_specs=[row_spec, w_spec, b_spec, w_spec, b_spec, w_spec],
        out_specs=[row_spec, row_spec, row_spec],
        out_shape=[out, out, out],
    )(feat, w_emb, b_emb, w_d, b1, w_s)


def _edge_body(spre, diff, wdist, ew2, eb2, xw1, xb1, xw2r, xb2, m2s, dw):
    d = diff[...]
    dist2 = jnp.sum(d * d, axis=1, keepdims=True)
    m = _silu(spre[...] + dist2 * wdist[...])
    m2 = _silu(_dot(m, ew2[...]) + eb2[...])
    t = _silu(_dot(m2, xw1[...]) + xb1[...])
    w = jnp.sum(t * xw2r[...], axis=1, keepdims=True) + xb2[...]
    m2s[...] = m2
    dw[...] = d * w


def _edge_call(spre, diff, wdist, ew2, eb2, xw1, xb1, xw2r, xb2):
    nb = N_EDGES // BLK_E
    row_spec = pl.BlockSpec((BLK_E, D), lambda i: (i, 0))
    x_spec = pl.BlockSpec((BLK_E, XPAD), lambda i: (i, 0))
    w_spec = pl.BlockSpec((D, D), lambda i: (0, 0))
    b_spec = pl.BlockSpec((1, D), lambda i: (0, 0))
    s_spec = pl.BlockSpec((1, 1), lambda i: (0, 0))
    return pl.pallas_call(
        _edge_body,
        grid=(nb,),
        in_specs=[row_spec, x_spec, b_spec, w_spec, b_spec, w_spec, b_spec,
                  b_spec, s_spec],
        out_specs=[row_spec, x_spec],
        out_shape=[jax.ShapeDtypeStruct((N_EDGES, D), jnp.float32),
                   jax.ShapeDtypeStruct((N_EDGES, XPAD), jnp.float32)],
    )(spre, diff, wdist, ew2, eb2, xw1, xb1, xw2r, xb2)


def _node_mid_body(h, xp, agg0, agg1, dx0, dx1, hw1a, hw1b, hb1, hw2, hb2,
                   w_d, b1n, w_s, h2o, xp2o, pre_d, pre_s):
    agg = (agg0[...] + agg1[...]) / 10.0
    u = _silu(_dot(h[...], hw1a[...]) + _dot(agg, hw1b[...]) + hb1[...])
    h2 = h[...] + _dot(u, hw2[...]) + hb2[...]
    h2o[...] = h2
    xp2o[...] = xp[...] + (dx0[...] + dx1[...]) / 10.0
    pre_d[...] = _dot(h2, w_d[...]) + b1n[...]
    pre_s[...] = _dot(h2, w_s[...])


def _node_mid_call(h, xp, agg0, agg1, dx0, dx1, hw1a, hw1b, hb1, hw2, hb2,
                   w_d, b1n, w_s):
    nb = N_NODES // BLK_N
    row_spec = pl.BlockSpec((BLK_N, D), lambda i: (i, 0))
    x_spec = pl.BlockSpec((BLK_N, XPAD), lambda i: (i, 0))
    w_spec = pl.BlockSpec((D, D), lambda i: (0, 0))
    b_spec = pl.BlockSpec((1, D), lambda i: (0, 0))
    outd = jax.ShapeDtypeStruct((N_NODES, D), jnp.float32)
    outx = jax.ShapeDtypeStruct((N_NODES, XPAD), jnp.float32)
    return pl.pallas_call(
        _node_mid_body,
        grid=(nb,),
        in_specs=[row_spec, x_spec, row_spec, row_spec, x_spec, x_spec,
                  w_spec, w_spec, b_spec, w_spec, b_spec,
                  w_spec, b_spec, w_spec],
        out_specs=[row_spec, x_spec, row_spec, row_spec],
        out_shape=[outd, outx, outd, outd],
    )(h, xp, agg0, agg1, dx0, dx1, hw1a, hw1b, hb1, hw2, hb2, w_d, b1n, w_s)


def _node_fin_body(h, xp, agg0, agg1, dx0, dx1, hw1a, hw1b, hb1, hw2, hb2,
                   w_out, b_out, outo, xp2o):
    agg = (agg0[...] + agg1[...]) / 10.0
    u = _silu(_dot(h[...], hw1a[...]) + _dot(agg, hw1b[...]) + hb1[...])
    h2 = h[...] + _dot(u, hw2[...]) + hb2[...]
    outo[...] = _dot(h2, w_out[...]) + b_out[...]
    xp2o[...] = xp[...] + (dx0[...] + dx1[...]) / 10.0


def _node_fin_call(h, xp, agg0, agg1, dx0, dx1, hw1a, hw1b, hb1, hw2, hb2,
                   w_out, b_out):
    nb = N_NODES // BLK_N
    row_spec = pl.BlockSpec((BLK_N, D), lambda i: (i, 0))
    x_spec = pl.BlockSpec((BLK_N, XPAD), lambda i: (i, 0))
    w_spec = pl.BlockSpec((D, D), lambda i: (0, 0))
    b_spec = pl.BlockSpec((1, D), lambda i: (0, 0))
    return pl.pallas_call(
        _node_fin_body,
        grid=(nb,),
        in_specs=[row_spec, x_spec, row_spec, row_spec, x_spec, x_spec,
                  w_spec, w_spec, b_spec, w_spec, b_spec, w_spec, b_spec],
        out_specs=[row_spec, x_spec],
        out_shape=[jax.ShapeDtypeStruct((N_NODES, D), jnp.float32),
                   jax.ShapeDtypeStruct((N_NODES, XPAD), jnp.float32)],
    )(h, xp, agg0, agg1, dx0, dx1, hw1a, hw1b, hb1, hw2, hb2, w_out, b_out)


_SC_MESH = plsc.VectorSubcoreMesh(core_axis_name="c", subcore_axis_name="s")
NW = 32
EPW = N_EDGES // NW
CH = 80
NCH = EPW // CH


def _wid():
    return lax.axis_index("s") * 2 + lax.axis_index("c")


_SC_PARAMS = pltpu.CompilerParams(use_tc_tiling_on_sc=False)


def _gth_issue(pre_d, pre_s, xp, src, dst, base, ids, idd, pd_b, ps_b,
               xd_b, xs_b, sem):
    pltpu.sync_copy(src.at[pl.ds(base, CH)], ids)
    pltpu.sync_copy(dst.at[pl.ds(base, CH)], idd)
    pltpu.async_copy(pre_d.at[idd], pd_b, sem)
    pltpu.async_copy(pre_s.at[ids], ps_b, sem)
    pltpu.async_copy(xp.at[idd], xd_b, sem)
    pltpu.async_copy(xp.at[ids], xs_b, sem)


def _gth_drain(pre_d, pre_s, xp, ids, idd, pd_b, ps_b, xd_b, xs_b, sem):
    pltpu.make_async_copy(pre_d.at[idd], pd_b, sem).wait()
    pltpu.make_async_copy(pre_s.at[ids], ps_b, sem).wait()
    pltpu.make_async_copy(xp.at[idd], xd_b, sem).wait()
    pltpu.make_async_copy(xp.at[ids], xs_b, sem).wait()


def _gth_compute_write(base, pd_b, ps_b, xd_b, xs_b, s_pre_o, diff_o):
    def row(i, _):
        for j in range(D // 16):
            sl = pl.ds(j * 16, 16)
            pd_b[i, sl] = pd_b[i, sl] + ps_b[i, sl]
        xd_b[i, :] = xd_b[i, :] - xs_b[i, :]
        return 0

    lax.fori_loop(0, CH, row, 0)
    pltpu.sync_copy(pd_b, s_pre_o.at[pl.ds(base, CH)])
    pltpu.sync_copy(xd_b, diff_o.at[pl.ds(base, CH)])


def _sc_gather_body(pre_d, pre_s, xp, src, dst, s_pre_o, diff_o,
                    ids_a, idd_a, pd_a, ps_a, xd_a, xs_a,
                    ids_b, idd_b, pd_b, ps_b, xd_b, xs_b, sem_a, sem_b):
    w = _wid()
    w0 = w * EPW
    seta = (ids_a, idd_a, pd_a, ps_a, xd_a, xs_a, sem_a)
    setb = (ids_b, idd_b, pd_b, ps_b, xd_b, xs_b, sem_b)

    _gth_issue(pre_d, pre_s, xp, src, dst, w0, *seta)

    def pair(u, _):
        a = 2 * u
        b = 2 * u + 1

        @pl.when(b < NCH)
        def _():
            _gth_issue(pre_d, pre_s, xp, src, dst, w0 + b * CH, *setb)

        _gth_drain(pre_d, pre_s, xp, *seta)
        _gth_compute_write(w0 + a * CH, pd_a, ps_a, xd_a, xs_a,
                           s_pre_o, diff_o)

        @pl.when(b + 1 < NCH)
        def _():
            _gth_issue(pre_d, pre_s, xp, src, dst, w0 + (b + 1) * CH, *seta)

        @pl.when(b < NCH)
        def _():
            _gth_drain(pre_d, pre_s, xp, *setb)
            _gth_compute_write(w0 + b * CH, pd_b, ps_b, xd_b, xs_b,
                               s_pre_o, diff_o)

        return 0

    lax.fori_loop(0, (NCH + 1) // 2, pair, 0)


@functools.partial(
    pl.kernel,
    out_type=[jax.ShapeDtypeStruct((N_EDGES, D), jnp.float32),
              jax.ShapeDtypeStruct((N_EDGES, XPAD), jnp.float32)],
    mesh=_SC_MESH,
    compiler_params=_SC_PARAMS,
    scratch_types=[
        pltpu.VMEM((CH,), jnp.int32),
        pltpu.VMEM((CH,), jnp.int32),
        pltpu.VMEM((CH, D), jnp.float32),
        pltpu.VMEM((CH, D), jnp.float32),
        pltpu.VMEM((CH, XPAD), jnp.float32),
        pltpu.VMEM((CH, XPAD), jnp.float32),
        pltpu.VMEM((CH,), jnp.int32),
        pltpu.VMEM((CH,), jnp.int32),
        pltpu.VMEM((CH, D), jnp.float32),
        pltpu.VMEM((CH, D), jnp.float32),
        pltpu.VMEM((CH, XPAD), jnp.float32),
        pltpu.VMEM((CH, XPAD), jnp.float32),
        pltpu.SemaphoreType.DMA,
        pltpu.SemaphoreType.DMA,
    ],
)
def _sc_gather_kernel(*args):
    _sc_gather_body(*args)


def _sc_gather(pre_d, pre_s, xp, src, dst):
    return _sc_gather_kernel(pre_d, pre_s, xp, src, dst)


NTILE = 16
EPC = N_EDGES // 2
EPT = EPC // NTILE
NRT = N_NODES // NTILE
ZR = 125


def _sc_scatter_body(m2, dw, dst, agg_o, dx_o,
                     idd, m2_b, dw_b, z128, z16, agg_sh, dx_sh, sem):
    cid = lax.axis_index("c")
    sid = lax.axis_index("s")

    def zrow(i, _):
        for j in range(D // 16):
            z128[i, pl.ds(j * 16, 16)] = jnp.zeros((16,), jnp.float32)
        z16[i, :] = jnp.zeros((16,), jnp.float32)
        return 0

    lax.fori_loop(0, ZR, zrow, 0)
    for k in range(NRT // ZR):
        off = sid * NRT + k * ZR
        pltpu.sync_copy(z128, agg_sh.at[pl.ds(off, ZR)])
        pltpu.sync_copy(z16, dx_sh.at[pl.ds(off, ZR)])
    plsc.subcore_barrier()

    def chunk(t, _):
        base = cid * EPC + sid * EPT + t * CH
        pltpu.sync_copy(dst.at[pl.ds(base, CH)], idd)
        cp1 = pltpu.async_copy(m2.at[pl.ds(base, CH)], m2_b, sem)
        cp2 = pltpu.async_copy(dw.at[pl.ds(base, CH)], dw_b, sem)
        cp1.wait()
        cp2.wait()
        pltpu.sync_copy(m2_b, agg_sh.at[idd], add=True)
        pltpu.sync_copy(dw_b, dx_sh.at[idd], add=True)
        return 0

    lax.fori_loop(0, EPT // CH, chunk, 0)
    plsc.subcore_barrier()

    for k in range(NRT // ZR):
        off = sid * NRT + k * ZR
        pltpu.sync_copy(agg_sh.at[pl.ds(off, ZR)], z128)
        pltpu.sync_copy(dx_sh.at[pl.ds(off, ZR)], z16)
        pltpu.sync_copy(z128, agg_o.at[cid, pl.ds(off, ZR)])
        pltpu.sync_copy(z16, dx_o.at[cid, pl.ds(off, ZR)])


@functools.partial(
    pl.kernel,
    out_type=[jax.ShapeDtypeStruct((2, N_NODES, D), jnp.float32),
              jax.ShapeDtypeStruct((2, N_NODES, XPAD), jnp.float32)],
    mesh=_SC_MESH,
    compiler_params=_SC_PARAMS,
    scratch_types=[
        pltpu.VMEM((CH,), jnp.int32),
        pltpu.VMEM((CH, D), jnp.float32),
        pltpu.VMEM((CH, XPAD), jnp.float32),
        pltpu.VMEM((ZR, D), jnp.float32),
        pltpu.VMEM((ZR, XPAD), jnp.float32),
        pltpu.VMEM_SHARED((N_NODES, D), jnp.float32),
        pltpu.VMEM_SHARED((N_NODES, XPAD), jnp.float32),
        pltpu.SemaphoreType.DMA,
    ],
)
def _sc_scatter_kernel(*args):
    _sc_scatter_body(*args)


def _sc_scatter(m2s, dw, dst):
    agg, dx = _sc_scatter_kernel(m2s, dw, dst)
    return agg[0], agg[1], dx[0], dx[1]


def kernel(feat, coordinate, edge_index, params):
    src = edge_index[0].astype(jnp.int32)
    dst = edge_index[1].astype(jnp.int32)
    xp = jnp.pad(coordinate, ((0, 0), (0, XPAD - 3)))

    lp = params['layers']

    def wparts(p):
        w1 = p['e_w1']
        return (w1[:D], w1[D:2 * D], w1[2 * D:2 * D + 1],
                p['e_b1'][None, :])

    w_d0, w_s0, _, b10 = wparts(lp[0])
    h, pre_d, pre_s = _entry_call(
        feat, params['emb_in_w'], params['emb_in_b'][None, :],
        w_d0, b10, w_s0)

    for li, p in enumerate(lp):
        _, _, wdist, _ = wparts(p)
        s_pre, diff = _sc_gather(pre_d, pre_s, xp, src, dst)
        m2s, dw = _edge_call(
            s_pre, diff, wdist, p['e_w2'], p['e_b2'][None, :],
            p['x_w1'], p['x_b1'][None, :], p['x_w2'].T, p['x_b2'][None, :])
        agg0, agg1, dx0, dx1 = _sc_scatter(m2s, dw, dst)
        hw1a, hw1b = p['h_w1'][:D], p['h_w1'][D:]
        if li + 1 < len(lp):
            w_dn, w_sn, _, b1n = wparts(lp[li + 1])
            h, xp, pre_d, pre_s = _node_mid_call(
                h, xp, agg0, agg1, dx0, dx1,
                hw1a, hw1b, p['h_b1'][None, :], p['h_w2'], p['h_b2'][None, :],
                w_dn, b1n, w_sn)
        else:
            out, xp = _node_fin_call(
                h, xp, agg0, agg1, dx0, dx1,
                hw1a, hw1b, p['h_b1'][None, :], p['h_w2'], p['h_b2'][None, :],
                params['emb_out_w'], params['emb_out_b'][None, :])

    return out, xp[:, :3]
